# Initial kernel scaffold; baseline (speedup 1.0000x reference)
import jax
import jax.numpy as jnp
from jax.experimental import pallas as pl


def kernel(x, edge_index, W1, b1, W2, b2, W3, b3):
    raise NotImplementedError("write your pallas kernel here")


# speedup vs baseline: 14.9885x; 14.9885x over previous
import functools

import jax
import jax.numpy as jnp
from jax import lax
from jax.experimental import pallas as pl
from jax.experimental.pallas import tpu as pltpu
from jax.experimental.pallas import tpu_sc as plsc

N = 10000
E = 320000
NW = 32
EPW = E // NW
K = 80
C = EPW // K
DW = 128

_MESH = plsc.VectorSubcoreMesh(core_axis_name="c", subcore_axis_name="s")


def _make_prop(F):
    @functools.partial(
        pl.kernel,
        mesh=_MESH,
        out_type=jax.ShapeDtypeStruct((2, N, F), jnp.float32),
        scratch_types=[
            pltpu.VMEM((C, K), jnp.int32),
            pltpu.VMEM((C, K), jnp.int32),
            pltpu.VMEM((K, F), jnp.float32),
            pltpu.VMEM_SHARED((N, F), jnp.float32),
            pltpu.SemaphoreType.DMA,
        ],
    )
    def prop(g_hbm, src_hbm, dst_hbm, zeros_hbm, out_hbm,
             src_v, dst_v, rows_v, acc, sem):
        cid = lax.axis_index("c")
        sid = lax.axis_index("s")
        wid = sid * 2 + cid

        @pl.when(sid == 0)
        def _():
            pltpu.sync_copy(zeros_hbm, acc)

        pltpu.sync_copy(src_hbm.at[wid], src_v)
        pltpu.sync_copy(dst_hbm.at[wid], dst_v)
        plsc.subcore_barrier()

        def body(i, carry):
            pltpu.async_copy(g_hbm.at[src_v.at[i]], rows_v, sem).wait()
            pltpu.sync_copy(rows_v, acc.at[dst_v.at[i]], add=True)
            return carry

        lax.fori_loop(0, C, body, 0)
        plsc.subcore_barrier()

        @pl.when(sid == 0)
        def _():
            pltpu.sync_copy(acc, out_hbm.at[cid])

    return prop


@functools.partial(
    pl.kernel,
    mesh=_MESH,
    out_type=jax.ShapeDtypeStruct((2, N, DW), jnp.float32),
    scratch_types=[
        pltpu.VMEM((C, K), jnp.int32),
        pltpu.VMEM((K, DW), jnp.float32),
        pltpu.VMEM_SHARED((N, DW), jnp.float32),
    ],
)
def _deg_kernel(dst_hbm, ones_hbm, zeros_hbm, out_hbm, dst_v, ones_v, acc):
    cid = lax.axis_index("c")
    sid = lax.axis_index("s")
    wid = sid * 2 + cid

    @pl.when(sid == 0)
    def _():
        pltpu.sync_copy(zeros_hbm, acc)

    pltpu.sync_copy(dst_hbm.at[wid], dst_v)
    pltpu.sync_copy(ones_hbm, ones_v)
    plsc.subcore_barrier()

    def body(i, carry):
        pltpu.sync_copy(ones_v, acc.at[dst_v.at[i]], add=True)
        return carry

    lax.fori_loop(0, C, body, 0)
    plsc.subcore_barrier()

    @pl.when(sid == 0)
    def _():
        pltpu.sync_copy(acc, out_hbm.at[cid])


_prop128 = _make_prop(128)

_GRID = 5
_BLK = N // _GRID


def _dinv_block(dega, degb):
    deg = dega[:, 0:1] + degb[:, 0:1] + 1.0
    dinv = lax.rsqrt(deg)
    return dinv


def _tc_first_body(dega_ref, degb_ref, x_ref, w_ref, t_ref, g_ref):
    dinv = _dinv_block(dega_ref[...], degb_ref[...])
    t = jnp.dot(x_ref[...], w_ref[...], preferred_element_type=jnp.float32)
    t_ref[...] = t
    g_ref[...] = t * dinv


def _tc_mid_body(dega_ref, degb_ref, sa_ref, sb_ref, t_ref, b_ref, w_ref,
                 tn_ref, gn_ref):
    dinv = _dinv_block(dega_ref[...], degb_ref[...])
    t = t_ref[...]
    p = dinv * (sa_ref[...] + sb_ref[...]) + (dinv * dinv) * t + b_ref[...]
    h = jnp.maximum(p, 0.0)
    tn = jnp.dot(h, w_ref[...], preferred_element_type=jnp.float32)
    tn_ref[...] = tn
    g = tn * dinv
    if gn_ref.shape[1] != tn.shape[1]:
        g = jnp.concatenate([g, jnp.zeros_like(g)], axis=1)
    gn_ref[...] = g


def _tc_last_body(dega_ref, degb_ref, sa_ref, sb_ref, t_ref, b_ref, o_ref):
    dinv = _dinv_block(dega_ref[...], degb_ref[...])
    p = dinv * (sa_ref[...] + sb_ref[...]) + (dinv * dinv) * t_ref[...] + b_ref[...]
    m = jnp.max(p, axis=1, keepdims=True)
    z = p - m
    lse = jnp.log(jnp.sum(jnp.exp(z), axis=1, keepdims=True))
    o_ref[...] = z - lse


def _row_spec(F):
    return pl.BlockSpec((_BLK, F), lambda i: (i, 0))


def _full_spec(shape):
    return pl.BlockSpec(shape, lambda i: (0,) * len(shape))


def _tc_first(dega, degb, x, w):
    return pl.pallas_call(
        _tc_first_body,
        grid=(_GRID,),
        in_specs=[_row_spec(DW), _row_spec(DW), _row_spec(128),
                  _full_spec((128, 128))],
        out_specs=[_row_spec(128), _row_spec(128)],
        out_shape=[jax.ShapeDtypeStruct((N, 128), jnp.float32)] * 2,
    )(dega, degb, x, w)


def _tc_mid(dega, degb, sa, sb, t, b, w, fout):
    gout = 128
    return pl.pallas_call(
        _tc_mid_body,
        grid=(_GRID,),
        in_specs=[_row_spec(DW), _row_spec(DW), _row_spec(128), _row_spec(128),
                  _row_spec(128), _full_spec((1, 128)),
                  _full_spec((128, fout))],
        out_specs=[_row_spec(fout), _row_spec(gout)],
        out_shape=[jax.ShapeDtypeStruct((N, fout), jnp.float32),
                   jax.ShapeDtypeStruct((N, gout), jnp.float32)],
    )(dega, degb, sa, sb, t, b, w)


def _tc_last(dega, degb, sa, sb, t, b):
    return pl.pallas_call(
        _tc_last_body,
        grid=(_GRID,),
        in_specs=[_row_spec(DW), _row_spec(DW), _row_spec(64), _row_spec(64),
                  _row_spec(64), _full_spec((1, 64))],
        out_specs=_row_spec(64),
        out_shape=jax.ShapeDtypeStruct((N, 64), jnp.float32),
    )(dega, degb, sa, sb, t, b)


def kernel(x, edge_index, W1, b1, W2, b2, W3, b3):
    ei = edge_index.astype(jnp.int32)
    src = ei[0].reshape(NW, C, K)
    dst = ei[1].reshape(NW, C, K)

    ones_rows = jnp.ones((K, DW), jnp.float32)
    z16 = jnp.zeros((N, DW), jnp.float32)
    z128 = jnp.zeros((N, 128), jnp.float32)

    degp = _deg_kernel(dst, ones_rows, z16)
    dega, degb = degp[0], degp[1]

    t1, g1 = _tc_first(dega, degb, x, W1)
    s1 = _prop128(g1, src, dst, z128)
    t2, g2 = _tc_mid(dega, degb, s1[0], s1[1], t1, b1.reshape(1, 128), W2, 128)
    s2 = _prop128(g2, src, dst, z128)
    t3, g3 = _tc_mid(dega, degb, s2[0], s2[1], t2, b2.reshape(1, 128), W3, 64)
    s3 = _prop128(g3, src, dst, z128)
    return _tc_last(dega, degb, s3[0, :, :64], s3[1, :, :64], t3,
                    b3.reshape(1, 64))


# speedup vs baseline: 21.0057x; 1.4015x over previous
import functools

import jax
import jax.numpy as jnp
from jax import lax
from jax.experimental import pallas as pl
from jax.experimental.pallas import tpu as pltpu
from jax.experimental.pallas import tpu_sc as plsc

N = 10000
E = 320000
NW = 32
EPW = E // NW
K = 80
C = EPW // K
TB = 5
CB = C // TB
DW = 128

_MESH = plsc.VectorSubcoreMesh(core_axis_name="c", subcore_axis_name="s")


def _make_prop(F):
    @functools.partial(
        pl.kernel,
        mesh=_MESH,
        out_type=jax.ShapeDtypeStruct((2, N, F), jnp.float32),
        scratch_types=[
            pltpu.VMEM((CB, K), jnp.int32),
            pltpu.VMEM((CB, K), jnp.int32),
            pltpu.VMEM((K, F), jnp.float32),
            pltpu.VMEM((K, F), jnp.float32),
            pltpu.VMEM_SHARED((N, F), jnp.float32),
            pltpu.SemaphoreType.DMA,
            pltpu.SemaphoreType.DMA,
        ],
    )
    def prop(g_hbm, src_hbm, dst_hbm, zeros_hbm, out_hbm,
             src_v, dst_v, rows0, rows1, acc, sem0, sem1):
        cid = lax.axis_index("c")
        sid = lax.axis_index("s")
        wid = sid * 2 + cid

        @pl.when(sid == 0)
        def _():
            pltpu.sync_copy(zeros_hbm, acc)

        def gstart(i, buf, sem):
            pltpu.async_copy(g_hbm.at[src_v.at[i]], buf, sem)

        def gwait(buf, sem):
            pltpu.make_async_copy(zeros_hbm.at[pl.ds(0, K)], buf, sem).wait()

        def scatter(i, buf):
            pltpu.sync_copy(buf, acc.at[dst_v.at[i]], add=True)

        plsc.subcore_barrier()

        def block(t, carry):
            blk = wid * TB + t
            pltpu.sync_copy(src_hbm.at[blk], src_v)
            pltpu.sync_copy(dst_hbm.at[blk], dst_v)
            gstart(0, rows0, sem0)

            def body(j, c2):
                i = 2 * j
                gstart(i + 1, rows1, sem1)
                gwait(rows0, sem0)
                scatter(i, rows0)
                gstart(i + 2, rows0, sem0)
                gwait(rows1, sem1)
                scatter(i + 1, rows1)
                return c2

            lax.fori_loop(0, (CB - 1) // 2, body, 0)
            gwait(rows0, sem0)
            scatter(CB - 1, rows0)
            return carry

        lax.fori_loop(0, TB, block, 0)

        plsc.subcore_barrier()

        @pl.when(sid == 0)
        def _():
            pltpu.sync_copy(acc, out_hbm.at[cid])

    return prop


@functools.partial(
    pl.kernel,
    mesh=_MESH,
    out_type=jax.ShapeDtypeStruct((2, N, DW), jnp.float32),
    scratch_types=[
        pltpu.VMEM((CB, K), jnp.int32),
        pltpu.VMEM((K, DW), jnp.float32),
        pltpu.VMEM_SHARED((N, DW), jnp.float32),
    ],
)
def _deg_kernel(dst_hbm, ones_hbm, zeros_hbm, out_hbm, dst_v, ones_v, acc):
    cid = lax.axis_index("c")
    sid = lax.axis_index("s")
    wid = sid * 2 + cid

    @pl.when(sid == 0)
    def _():
        pltpu.sync_copy(zeros_hbm, acc)

    pltpu.sync_copy(ones_hbm, ones_v)
    plsc.subcore_barrier()

    def block(t, carry):
        pltpu.sync_copy(dst_hbm.at[wid * TB + t], dst_v)

        def body(i, c2):
            pltpu.sync_copy(ones_v, acc.at[dst_v.at[i]], add=True)
            return c2

        lax.fori_loop(0, CB, body, 0)
        return carry

    lax.fori_loop(0, TB, block, 0)
    plsc.subcore_barrier()

    @pl.when(sid == 0)
    def _():
        pltpu.sync_copy(acc, out_hbm.at[cid])


_prop128 = _make_prop(128)

_GRID = 5
_BLK = N // _GRID


def _dinv_block(dega, degb):
    deg = dega[:, 0:1] + degb[:, 0:1] + 1.0
    dinv = lax.rsqrt(deg)
    return dinv


def _tc_first_body(dega_ref, degb_ref, x_ref, w_ref, t_ref, g_ref):
    dinv = _dinv_block(dega_ref[...], degb_ref[...])
    t = jnp.dot(x_ref[...], w_ref[...], preferred_element_type=jnp.float32)
    t_ref[...] = t
    g_ref[...] = t * dinv


def _tc_mid_body(dega_ref, degb_ref, sa_ref, sb_ref, t_ref, b_ref, w_ref,
                 tn_ref, gn_ref):
    dinv = _dinv_block(dega_ref[...], degb_ref[...])
    t = t_ref[...]
    p = dinv * (sa_ref[...] + sb_ref[...]) + (dinv * dinv) * t + b_ref[...]
    h = jnp.maximum(p, 0.0)
    tn = jnp.dot(h, w_ref[...], preferred_element_type=jnp.float32)
    tn_ref[...] = tn
    g = tn * dinv
    if gn_ref.shape[1] != tn.shape[1]:
        g = jnp.concatenate([g, jnp.zeros_like(g)], axis=1)
    gn_ref[...] = g


def _tc_last_body(dega_ref, degb_ref, sa_ref, sb_ref, t_ref, b_ref, o_ref):
    dinv = _dinv_block(dega_ref[...], degb_ref[...])
    p = dinv * (sa_ref[...] + sb_ref[...]) + (dinv * dinv) * t_ref[...] + b_ref[...]
    m = jnp.max(p, axis=1, keepdims=True)
    z = p - m
    lse = jnp.log(jnp.sum(jnp.exp(z), axis=1, keepdims=True))
    o_ref[...] = z - lse


def _row_spec(F):
    return pl.BlockSpec((_BLK, F), lambda i: (i, 0))


def _full_spec(shape):
    return pl.BlockSpec(shape, lambda i: (0,) * len(shape))


def _tc_first(dega, degb, x, w):
    return pl.pallas_call(
        _tc_first_body,
        grid=(_GRID,),
        in_specs=[_row_spec(DW), _row_spec(DW), _row_spec(128),
                  _full_spec((128, 128))],
        out_specs=[_row_spec(128), _row_spec(128)],
        out_shape=[jax.ShapeDtypeStruct((N, 128), jnp.float32)] * 2,
    )(dega, degb, x, w)


def _tc_mid(dega, degb, sa, sb, t, b, w, fout):
    gout = 128
    return pl.pallas_call(
        _tc_mid_body,
        grid=(_GRID,),
        in_specs=[_row_spec(DW), _row_spec(DW), _row_spec(128), _row_spec(128),
                  _row_spec(128), _full_spec((1, 128)),
                  _full_spec((128, fout))],
        out_specs=[_row_spec(fout), _row_spec(gout)],
        out_shape=[jax.ShapeDtypeStruct((N, fout), jnp.float32),
                   jax.ShapeDtypeStruct((N, gout), jnp.float32)],
    )(dega, degb, sa, sb, t, b, w)


def _tc_last(dega, degb, sa, sb, t, b):
    return pl.pallas_call(
        _tc_last_body,
        grid=(_GRID,),
        in_specs=[_row_spec(DW), _row_spec(DW), _row_spec(64), _row_spec(64),
                  _row_spec(64), _full_spec((1, 64))],
        out_specs=_row_spec(64),
        out_shape=jax.ShapeDtypeStruct((N, 64), jnp.float32),
    )(dega, degb, sa, sb, t, b)


def kernel(x, edge_index, W1, b1, W2, b2, W3, b3):
    ei = edge_index.astype(jnp.int32)
    src = ei[0].reshape(NW * TB, CB, K)
    dst = ei[1].reshape(NW * TB, CB, K)

    ones_rows = jnp.ones((K, DW), jnp.float32)
    z16 = jnp.zeros((N, DW), jnp.float32)
    z128 = jnp.zeros((N, 128), jnp.float32)

    degp = _deg_kernel(dst, ones_rows, z16)
    dega, degb = degp[0], degp[1]

    t1, g1 = _tc_first(dega, degb, x, W1)
    s1 = _prop128(g1, src, dst, z128)
    t2, g2 = _tc_mid(dega, degb, s1[0], s1[1], t1, b1.reshape(1, 128), W2, 128)
    s2 = _prop128(g2, src, dst, z128)
    t3, g3 = _tc_mid(dega, degb, s2[0], s2[1], t2, b2.reshape(1, 128), W3, 64)
    s3 = _prop128(g3, src, dst, z128)
    return _tc_last(dega, degb, s3[0, :, :64], s3[1, :, :64], t3,
                    b3.reshape(1, 64))


# speedup vs baseline: 23.3349x; 1.1109x over previous
import functools

import jax
import jax.numpy as jnp
from jax import lax
from jax.experimental import pallas as pl
from jax.experimental.pallas import tpu as pltpu
from jax.experimental.pallas import tpu_sc as plsc

N = 10000
E = 320000
NW = 32
EPW = E // NW
K = 80
C = EPW // K
TB = 5
CB = C // TB
DW = 128

_MESH = plsc.VectorSubcoreMesh(core_axis_name="c", subcore_axis_name="s")


def _make_prop(F):
    @functools.partial(
        pl.kernel,
        mesh=_MESH,
        out_type=jax.ShapeDtypeStruct((2, N, F), jnp.float32),
        scratch_types=[
            pltpu.VMEM((CB, K), jnp.int32),
            pltpu.VMEM((CB, K), jnp.int32),
            pltpu.VMEM((K, F), jnp.float32),
            pltpu.VMEM((K, F), jnp.float32),
            pltpu.VMEM((K, F), jnp.float32),
            pltpu.VMEM_SHARED((N, F), jnp.float32),
            pltpu.SemaphoreType.DMA,
            pltpu.SemaphoreType.DMA,
            pltpu.SemaphoreType.DMA,
        ],
    )
    def prop(g_hbm, src_hbm, dst_hbm, zeros_hbm, out_hbm,
             src_v, dst_v, rows0, rows1, rows2, acc, sem0, sem1, sem2):
        cid = lax.axis_index("c")
        sid = lax.axis_index("s")
        wid = sid * 2 + cid

        @pl.when(sid == 0)
        def _():
            pltpu.sync_copy(zeros_hbm, acc)

        def gstart(i, buf, sem):
            pltpu.async_copy(g_hbm.at[src_v.at[i]], buf, sem)

        def gwait(buf, sem):
            pltpu.make_async_copy(zeros_hbm.at[pl.ds(0, K)], buf, sem).wait()

        def scatter(i, buf):
            pltpu.sync_copy(buf, acc.at[dst_v.at[i]], add=True)

        plsc.subcore_barrier()

        def block(t, carry):
            blk = wid * TB + t
            pltpu.sync_copy(src_hbm.at[blk], src_v)
            pltpu.sync_copy(dst_hbm.at[blk], dst_v)
            gstart(0, rows0, sem0)
            gstart(1, rows1, sem1)

            def body(j, c2):
                i = 3 * j
                gstart(i + 2, rows2, sem2)
                gwait(rows0, sem0)
                scatter(i, rows0)
                gstart(i + 3, rows0, sem0)
                gwait(rows1, sem1)
                scatter(i + 1, rows1)

                @pl.when(i + 4 < CB)
                def _():
                    gstart(i + 4, rows1, sem1)

                gwait(rows2, sem2)
                scatter(i + 2, rows2)
                return c2

            lax.fori_loop(0, CB // 3, body, 0)
            gwait(rows0, sem0)
            scatter(CB - 1, rows0)
            return carry

        lax.fori_loop(0, TB, block, 0)

        plsc.subcore_barrier()

        @pl.when(sid == 0)
        def _():
            pltpu.sync_copy(acc, out_hbm.at[cid])

    return prop


@functools.partial(
    pl.kernel,
    mesh=_MESH,
    out_type=jax.ShapeDtypeStruct((2, N, DW), jnp.float32),
    scratch_types=[
        pltpu.VMEM((CB, K), jnp.int32),
        pltpu.VMEM((K, DW), jnp.float32),
        pltpu.VMEM_SHARED((N, DW), jnp.float32),
        pltpu.SemaphoreType.DMA,
    ],
)
def _deg_kernel(dst_hbm, ones_hbm, zeros_hbm, out_hbm, dst_v, ones_v, acc, sem):
    cid = lax.axis_index("c")
    sid = lax.axis_index("s")
    wid = sid * 2 + cid

    @pl.when(sid == 0)
    def _():
        pltpu.sync_copy(zeros_hbm, acc)

    pltpu.sync_copy(ones_hbm, ones_v)
    plsc.subcore_barrier()

    def block(t, carry):
        pltpu.sync_copy(dst_hbm.at[wid * TB + t], dst_v)

        def body(i, c2):
            pltpu.async_copy(ones_v, acc.at[dst_v.at[i]], sem, add=True)
            return c2

        lax.fori_loop(0, CB, body, 0)

        def drain(i, c2):
            pltpu.make_async_copy(zeros_hbm.at[pl.ds(0, K)], ones_v, sem).wait()
            return c2

        lax.fori_loop(0, CB, drain, 0)
        return carry

    lax.fori_loop(0, TB, block, 0)
    plsc.subcore_barrier()

    @pl.when(sid == 0)
    def _():
        pltpu.sync_copy(acc, out_hbm.at[cid])


_prop128 = _make_prop(128)

_GRID = 5
_BLK = N // _GRID


def _dinv_block(dega, degb):
    deg = dega[:, 0:1] + degb[:, 0:1] + 1.0
    dinv = lax.rsqrt(deg)
    return dinv


def _tc_first_body(dega_ref, degb_ref, x_ref, w_ref, t_ref, g_ref):
    dinv = _dinv_block(dega_ref[...], degb_ref[...])
    t = jnp.dot(x_ref[...], w_ref[...], preferred_element_type=jnp.float32)
    t_ref[...] = t
    g_ref[...] = t * dinv


def _tc_mid_body(dega_ref, degb_ref, sa_ref, sb_ref, t_ref, b_ref, w_ref,
                 tn_ref, gn_ref):
    dinv = _dinv_block(dega_ref[...], degb_ref[...])
    t = t_ref[...]
    p = dinv * (sa_ref[...] + sb_ref[...]) + (dinv * dinv) * t + b_ref[...]
    h = jnp.maximum(p, 0.0)
    tn = jnp.dot(h, w_ref[...], preferred_element_type=jnp.float32)
    tn_ref[...] = tn
    g = tn * dinv
    if gn_ref.shape[1] != tn.shape[1]:
        g = jnp.concatenate([g, jnp.zeros_like(g)], axis=1)
    gn_ref[...] = g


def _tc_last_body(dega_ref, degb_ref, sa_ref, sb_ref, t_ref, b_ref, o_ref):
    dinv = _dinv_block(dega_ref[...], degb_ref[...])
    p = dinv * (sa_ref[...] + sb_ref[...]) + (dinv * dinv) * t_ref[...] + b_ref[...]
    m = jnp.max(p, axis=1, keepdims=True)
    z = p - m
    lse = jnp.log(jnp.sum(jnp.exp(z), axis=1, keepdims=True))
    o_ref[...] = z - lse


def _row_spec(F):
    return pl.BlockSpec((_BLK, F), lambda i: (i, 0))


def _full_spec(shape):
    return pl.BlockSpec(shape, lambda i: (0,) * len(shape))


def _tc_first(dega, degb, x, w):
    return pl.pallas_call(
        _tc_first_body,
        grid=(_GRID,),
        in_specs=[_row_spec(DW), _row_spec(DW), _row_spec(128),
                  _full_spec((128, 128))],
        out_specs=[_row_spec(128), _row_spec(128)],
        out_shape=[jax.ShapeDtypeStruct((N, 128), jnp.float32)] * 2,
    )(dega, degb, x, w)


def _tc_mid(dega, degb, sa, sb, t, b, w, fout):
    gout = 128
    return pl.pallas_call(
        _tc_mid_body,
        grid=(_GRID,),
        in_specs=[_row_spec(DW), _row_spec(DW), _row_spec(128), _row_spec(128),
                  _row_spec(128), _full_spec((1, 128)),
                  _full_spec((128, fout))],
        out_specs=[_row_spec(fout), _row_spec(gout)],
        out_shape=[jax.ShapeDtypeStruct((N, fout), jnp.float32),
                   jax.ShapeDtypeStruct((N, gout), jnp.float32)],
    )(dega, degb, sa, sb, t, b, w)


def _tc_last(dega, degb, sa, sb, t, b):
    return pl.pallas_call(
        _tc_last_body,
        grid=(_GRID,),
        in_specs=[_row_spec(DW), _row_spec(DW), _row_spec(64), _row_spec(64),
                  _row_spec(64), _full_spec((1, 64))],
        out_specs=_row_spec(64),
        out_shape=jax.ShapeDtypeStruct((N, 64), jnp.float32),
    )(dega, degb, sa, sb, t, b)


def kernel(x, edge_index, W1, b1, W2, b2, W3, b3):
    ei = edge_index.astype(jnp.int32)
    src = ei[0].reshape(NW * TB, CB, K)
    dst = ei[1].reshape(NW * TB, CB, K)

    ones_rows = jnp.ones((K, DW), jnp.float32)
    z16 = jnp.zeros((N, DW), jnp.float32)
    z128 = jnp.zeros((N, 128), jnp.float32)

    degp = _deg_kernel(dst, ones_rows, z16)
    dega, degb = degp[0], degp[1]

    t1, g1 = _tc_first(dega, degb, x, W1)
    s1 = _prop128(g1, src, dst, z128)
    t2, g2 = _tc_mid(dega, degb, s1[0], s1[1], t1, b1.reshape(1, 128), W2, 128)
    s2 = _prop128(g2, src, dst, z128)
    t3, g3 = _tc_mid(dega, degb, s2[0], s2[1], t2, b2.reshape(1, 128), W3, 64)
    s3 = _prop128(g3, src, dst, z128)
    return _tc_last(dega, degb, s3[0, :, :64], s3[1, :, :64], t3,
                    b3.reshape(1, 64))
